# baseline (speedup 1.0000x reference)
import jax
import jax.numpy as jnp
from jax import lax
from jax.experimental import pallas as pl
from jax.experimental.pallas import tpu as pltpu
from jax.experimental.pallas import tpu_sc as plsc

N = 10000
NP = 10240
D_IN = 128
DH = 32
D_OUT = 128
E = 320000
NC, NS, K = 2, 16, 128
NW = NC * NS
CH = 80
EP = NW * CH * K
RPS = NP // NS
NB = 8

_f32 = jnp.float32
_mesh = plsc.VectorSubcoreMesh(core_axis_name="c", subcore_axis_name="s")
_sc_params = pltpu.CompilerParams(use_tc_tiling_on_sc=False)


ECH = E // K
CB = ECH // NW
CR = ECH - CB * NW


def _deg_body(edge_hbm, zeros_hbm, ones_hbm, out_hbm, idx_v, ones_v, acc_sh,
              dsem):
    cid = lax.axis_index("c")
    sid = lax.axis_index("s")
    wid = sid * NC + cid
    row0 = sid * RPS
    bw = CB * wid + jnp.minimum(wid, CR)
    cw = CB + (wid < CR).astype(jnp.int32)
    pltpu.sync_copy(zeros_hbm.at[pl.ds(row0, RPS)], acc_sh.at[pl.ds(row0, RPS)])
    pltpu.sync_copy(ones_hbm, ones_v)
    pltpu.sync_copy(edge_hbm.at[1, pl.ds(bw, CB)], idx_v.at[pl.ds(0, CB)])

    @pl.when(wid < CR)
    def _():
        pltpu.sync_copy(edge_hbm.at[1, bw + CB], idx_v.at[CB])

    plsc.subcore_barrier()

    @pl.loop(0, cw)
    def _(c):
        pltpu.async_copy(ones_v, acc_sh.at[idx_v.at[c]], dsem, add=True)

    @pl.loop(0, cw)
    def _(c):
        pltpu.make_async_copy(ones_v, acc_sh.at[idx_v.at[c]], dsem).wait()

    plsc.subcore_barrier()
    pltpu.sync_copy(acc_sh.at[pl.ds(row0, RPS)],
                    out_hbm.at[cid, pl.ds(row0, RPS)])


def _sc_degree(edge_r, zeros16, ones16):
    return pl.kernel(
        _deg_body,
        out_type=jax.ShapeDtypeStruct((NC, NP, 16), _f32),
        mesh=_mesh,
        scratch_types=[
            pltpu.VMEM((CB + 1, K), jnp.int32),
            pltpu.VMEM((K, 16), _f32),
            pltpu.VMEM_SHARED((NP, 16), _f32),
            pltpu.SemaphoreType.DMA,
        ],
        compiler_params=_sc_params,
    )(edge_r, zeros16, ones16)


def _rsqrt_nr(x):
    i = lax.bitcast_convert_type(x, jnp.int32)
    i = 0x5F3759DF - lax.shift_right_logical(i, 1)
    y = lax.bitcast_convert_type(i, _f32)
    for _ in range(3):
        y = y * (1.5 - 0.5 * x * y * y)
    return y


def _pipeline(src_v, dst_v, rows, y_sh, acc_sh, gsem, ssem, nb):
    def gather_start(c, b):
        pltpu.async_copy(y_sh.at[src_v.at[c]], rows.at[b], gsem.at[b])

    def gather_wait(c, b):
        pltpu.make_async_copy(y_sh.at[src_v.at[c]], rows.at[b],
                              gsem.at[b]).wait()

    def scat_start(c, b):
        pltpu.async_copy(rows.at[b], acc_sh.at[dst_v.at[c]], ssem.at[b],
                         add=True)

    def scat_wait(c, b):
        pltpu.make_async_copy(rows.at[b], acc_sh.at[dst_v.at[c]],
                              ssem.at[b]).wait()

    for b in range(nb):
        gather_start(b, b)

    ng = CH // nb

    @pl.loop(0, ng)
    def _(i):
        c0 = i * nb
        for b in range(nb):
            gather_wait(c0 + b, b)
            scat_start(c0 + b, b)
            scat_wait(c0 + b, b)

            @pl.when(i < ng - 1)
            def _():
                gather_start(c0 + nb + b, b)


NBU = 4


BP = 128
NBLK = RPS // BP


def _agg_body(xw_hbm, deg_hbm, a_hbm, b_hbm, lo_hbm, prm_hbm,
              src_hbm, dst_hbm, zeros_hbm, out_hbm, tab_hbm,
              src_v, dst_v, xw_v, d0_v, d1_v, a0_v, a1_v, b_v, lo_v, prm_v,
              rows, y_sh, acc_sh, gsem, ssem):
    cid = lax.axis_index("c")
    sid = lax.axis_index("s")
    wid = sid * NC + cid
    row0 = sid * RPS
    pltpu.sync_copy(zeros_hbm.at[pl.ds(row0, RPS)], acc_sh.at[pl.ds(row0, RPS)])
    pltpu.sync_copy(b_hbm, b_v)
    pltpu.sync_copy(lo_hbm, lo_v)
    pltpu.sync_copy(prm_hbm, prm_v)
    pltpu.sync_copy(src_hbm.at[wid], src_v)
    pltpu.sync_copy(dst_hbm.at[wid], dst_v)

    c1 = prm_v.at[pl.ds(0, 1), pl.ds(0, 16)][...]
    c2 = prm_v.at[pl.ds(1, 1), pl.ds(0, 16)][...]
    c2p = prm_v.at[pl.ds(2, 1), pl.ds(0, 16)][...]
    c3 = prm_v.at[pl.ds(3, 1), pl.ds(0, 16)][...]
    c3p = prm_v.at[pl.ds(4, 1), pl.ds(0, 16)][...]

    @pl.loop(0, NBLK)
    def _(blk):
        r0 = row0 + blk * BP
        pltpu.sync_copy(xw_hbm.at[pl.ds(r0, BP)], xw_v)
        pltpu.sync_copy(deg_hbm.at[0, pl.ds(r0, BP)], d0_v)
        pltpu.sync_copy(deg_hbm.at[1, pl.ds(r0, BP)], d1_v)
        pltpu.sync_copy(a_hbm.at[0, pl.ds(r0, BP)], a0_v)
        pltpu.sync_copy(a_hbm.at[1, pl.ds(r0, BP)], a1_v)

        @pl.loop(0, BP)
        def _(r):
            rs = pl.ds(r, 1)
            dv = (d0_v.at[rs, pl.ds(0, 16)][...]
                  + d1_v.at[rs, pl.ds(0, 16)][...])
            dis = _rsqrt_nr(dv + 1.0)
            ac = c1 * dis
            xwc = (c2 * dis + c2p) * dis
            outc = c3 * dis + c3p
            for h in (0, 16):
                hs = (rs, pl.ds(h, 16))
                inner = (ac * (a0_v.at[*hs][...] + a1_v.at[*hs][...])
                         + xwc * xw_v.at[*hs][...]
                         + b_v.at[pl.ds(0, 1), pl.ds(h, 16)][...])
                inner = jnp.maximum(
                    inner, lo_v.at[pl.ds(0, 1), pl.ds(h, 16)][...])
                xw_v.at[*hs][...] = outc * inner

        pltpu.sync_copy(xw_v, y_sh.at[pl.ds(r0, BP)])

        @pl.when(cid == 0)
        def _():
            pltpu.sync_copy(xw_v, tab_hbm.at[pl.ds(r0, BP)])

    plsc.subcore_barrier()
    _pipeline(src_v, dst_v, rows, y_sh, acc_sh, gsem, ssem, NBU)
    plsc.subcore_barrier()
    pltpu.sync_copy(acc_sh.at[pl.ds(row0, RPS)],
                    out_hbm.at[cid, pl.ds(row0, RPS)])


def _sc_agg(xw, degacc, a, b, lo, prm, src_r, dst_r, zeros32):
    return pl.kernel(
        _agg_body,
        out_type=[jax.ShapeDtypeStruct((NC, NP, DH), _f32),
                  jax.ShapeDtypeStruct((NP, DH), _f32)],
        mesh=_mesh,
        scratch_types=[
            pltpu.VMEM((CH, K), jnp.int32),
            pltpu.VMEM((CH, K), jnp.int32),
            pltpu.VMEM((BP, DH), _f32),
            pltpu.VMEM((BP, 16), _f32),
            pltpu.VMEM((BP, 16), _f32),
            pltpu.VMEM((BP, DH), _f32),
            pltpu.VMEM((BP, DH), _f32),
            pltpu.VMEM((1, DH), _f32),
            pltpu.VMEM((1, DH), _f32),
            pltpu.VMEM((8, 16), _f32),
            pltpu.VMEM((NBU, K, DH), _f32),
            pltpu.VMEM_SHARED((NP, DH), _f32),
            pltpu.VMEM_SHARED((NP, DH), _f32),
            pltpu.SemaphoreType.DMA((NBU,)),
            pltpu.SemaphoreType.DMA((NBU,)),
        ],
        compiler_params=_sc_params,
    )(xw, degacc, a, b, lo, prm, src_r, dst_r, zeros32)


def _mm1_body(x_ref, w_ref, o_ref):
    o_ref[...] = jnp.dot(x_ref[...], w_ref[...], preferred_element_type=_f32)


def _copy_body(i_ref, o_ref):
    o_ref[...] = i_ref[...]


def _final_body(acc_ref, y2_ref, degacc_ref, w_ref, b_ref, y1_ref, o_ref):
    del y1_ref
    deg = degacc_ref[0] + degacc_ref[1] + 1.0
    dis = lax.rsqrt(deg)[:, 0:1]
    agg = dis * (acc_ref[0] + acc_ref[1] + y2_ref[...])
    z = jnp.dot(agg, w_ref[...], preferred_element_type=_f32) + b_ref[...]
    o_ref[...] = jax.nn.sigmoid(z)


def kernel(x, edge_index, W1, b1, W2, b2):
    src = edge_index[0]
    dst = edge_index[1]
    pad = EP - E
    src_r = jnp.concatenate(
        [src, jnp.zeros((pad,), jnp.int32)]).reshape(NW, CH, K)
    dst_r = jnp.concatenate(
        [dst, jnp.full((pad,), N, jnp.int32)]).reshape(NW, CH, K)
    x_p = jnp.pad(x, ((0, NP - N), (0, 0)))
    zeros16 = jnp.zeros((NP, 16), _f32)
    ones16 = jnp.ones((K, 16), _f32)
    zeros32 = jnp.zeros((NP, DH), _f32)
    zeros2 = jnp.zeros((NC, NP, DH), _f32)
    zrow = jnp.zeros((1, DH), _f32)
    ninf = jnp.full((1, DH), -3.0e38, _f32)
    b1r = b1.reshape(1, DH)
    b2r = b2.reshape(1, D_OUT)
    edge_r = edge_index.reshape(2, ECH, K)

    def prm(c1, c2, c2p, c3, c3p):
        v = jnp.array([c1, c2, c2p, c3, c3p, 0.0, 0.0, 0.0], _f32)
        return jnp.broadcast_to(v[:, None], (8, 16))

    degacc = _sc_degree(edge_r, zeros16, ones16)
    xw = pl.pallas_call(
        _mm1_body,
        out_shape=jax.ShapeDtypeStruct((NP, DH), _f32),
    )(x_p, W1)

    acc1, y1 = _sc_agg(xw, degacc, zeros2, zrow, ninf,
                       prm(0.0, 0.0, 1.0, 0.0, 1.0), src_r, dst_r, zeros32)
    acc1 = pl.pallas_call(
        _copy_body,
        out_shape=jax.ShapeDtypeStruct((NC, NP, DH), _f32),
    )(acc1)
    acc2, y2 = _sc_agg(xw, degacc, acc1, b1r, zrow,
                       prm(1.0, 1.0, 0.0, 1.0, 0.0), src_r, dst_r, zeros32)

    out = pl.pallas_call(
        _final_body,
        out_shape=jax.ShapeDtypeStruct((NP, D_OUT), _f32),
    )(acc2, y2, degacc, W2, b2r, y1)

    return out[:N]


# speedup vs baseline: 1.4885x; 1.4885x over previous
import jax
import jax.numpy as jnp
from jax import lax
from jax.experimental import pallas as pl
from jax.experimental.pallas import tpu as pltpu
from jax.experimental.pallas import tpu_sc as plsc

N = 10000
NP = 10240
D_IN = 128
DH = 32
D_OUT = 128
E = 320000
NC, NS, K = 2, 16, 128
NW = NC * NS
RPS = NP // NS
NB = 6
ECH = E // K
CB = ECH // NW
CR = ECH - CB * NW

_f32 = jnp.float32
_mesh = plsc.VectorSubcoreMesh(core_axis_name="c", subcore_axis_name="s")
_sc_params = pltpu.CompilerParams(use_tc_tiling_on_sc=False)


def _slab(wid):
    return CB * wid + jnp.minimum(wid, CR)


def _deg_body(edge_hbm, zeros_hbm, ones_hbm, out_hbm, idx_v, ones_v, acc_sh,
              dsem):
    cid = lax.axis_index("c")
    sid = lax.axis_index("s")
    wid = sid * NC + cid
    row0 = sid * RPS
    bw = _slab(wid)
    cw = CB + (wid < CR).astype(jnp.int32)
    pltpu.sync_copy(zeros_hbm.at[pl.ds(row0, RPS)], acc_sh.at[pl.ds(row0, RPS)])
    pltpu.sync_copy(ones_hbm, ones_v)
    pltpu.sync_copy(edge_hbm.at[1, pl.ds(bw, CB)], idx_v.at[pl.ds(0, CB)])

    @pl.when(wid < CR)
    def _():
        pltpu.sync_copy(edge_hbm.at[1, bw + CB], idx_v.at[CB])

    plsc.subcore_barrier()

    @pl.loop(0, cw)
    def _(c):
        pltpu.async_copy(ones_v, acc_sh.at[idx_v.at[c]], dsem, add=True)

    @pl.loop(0, cw)
    def _(c):
        pltpu.make_async_copy(ones_v, acc_sh.at[idx_v.at[c]], dsem).wait()

    plsc.subcore_barrier()
    pltpu.sync_copy(acc_sh.at[pl.ds(row0, RPS)],
                    out_hbm.at[cid, pl.ds(row0, RPS)])


def _sc_degree(edge_r, zeros16, ones16):
    return pl.kernel(
        _deg_body,
        out_type=jax.ShapeDtypeStruct((NC, NP, 16), _f32),
        mesh=_mesh,
        scratch_types=[
            pltpu.VMEM((CB + 1, K), jnp.int32),
            pltpu.VMEM((K, 16), _f32),
            pltpu.VMEM_SHARED((NP, 16), _f32),
            pltpu.SemaphoreType.DMA,
        ],
        compiler_params=_sc_params,
    )(edge_r, zeros16, ones16)


def _agg_body(y_hbm, edge_hbm, zeros_hbm, out_hbm,
              src_v, dst_v, rows, y_sh, acc_sh, gsem, ssem):
    cid = lax.axis_index("c")
    sid = lax.axis_index("s")
    wid = sid * NC + cid
    row0 = sid * RPS
    bw = _slab(wid)
    pltpu.sync_copy(zeros_hbm.at[pl.ds(row0, RPS)], acc_sh.at[pl.ds(row0, RPS)])
    pltpu.sync_copy(y_hbm.at[pl.ds(row0, RPS)], y_sh.at[pl.ds(row0, RPS)])
    pltpu.sync_copy(edge_hbm.at[0, pl.ds(bw, CB)], src_v.at[pl.ds(0, CB)])
    pltpu.sync_copy(edge_hbm.at[1, pl.ds(bw, CB)], dst_v.at[pl.ds(0, CB)])

    @pl.when(wid < CR)
    def _():
        pltpu.sync_copy(edge_hbm.at[0, bw + CB], src_v.at[CB])
        pltpu.sync_copy(edge_hbm.at[1, bw + CB], dst_v.at[CB])

    plsc.subcore_barrier()

    def gather_start(c, b):
        pltpu.async_copy(y_sh.at[src_v.at[c]], rows.at[b], gsem.at[b])

    def gather_wait(c, b):
        pltpu.make_async_copy(y_sh.at[src_v.at[c]], rows.at[b],
                              gsem.at[b]).wait()

    def scat_start(c, b):
        pltpu.async_copy(rows.at[b], acc_sh.at[dst_v.at[c]], ssem.at[b],
                         add=True)

    def scat_wait(c, b):
        pltpu.make_async_copy(rows.at[b], acc_sh.at[dst_v.at[c]],
                              ssem.at[b]).wait()

    for b in range(NB):
        gather_start(b, b)

    NG = CB // NB

    @pl.loop(0, NG)
    def _(i):
        c0 = i * NB
        for b in range(NB):
            gather_wait(c0 + b, b)
            scat_start(c0 + b, b)
            scat_wait(c0 + b, b)

            @pl.when(i < NG - 1)
            def _():
                gather_start(c0 + NB + b, b)

    @pl.when(wid < CR)
    def _():
        pltpu.sync_copy(y_sh.at[src_v.at[CB]], rows.at[0])
        pltpu.sync_copy(rows.at[0], acc_sh.at[dst_v.at[CB]], add=True)

    plsc.subcore_barrier()
    pltpu.sync_copy(acc_sh.at[pl.ds(row0, RPS)],
                    out_hbm.at[cid, pl.ds(row0, RPS)])


def _sc_aggregate(y, edge_r, zeros32):
    return pl.kernel(
        _agg_body,
        out_type=jax.ShapeDtypeStruct((NC, NP, DH), _f32),
        mesh=_mesh,
        scratch_types=[
            pltpu.VMEM((CB + 1, K), jnp.int32),
            pltpu.VMEM((CB + 1, K), jnp.int32),
            pltpu.VMEM((NB, K, DH), _f32),
            pltpu.VMEM_SHARED((NP, DH), _f32),
            pltpu.VMEM_SHARED((NP, DH), _f32),
            pltpu.SemaphoreType.DMA((NB,)),
            pltpu.SemaphoreType.DMA((NB,)),
        ],
        compiler_params=_sc_params,
    )(y, edge_r, zeros32)


def _mm1_body(x_ref, w_ref, o_ref):
    o_ref[...] = jnp.dot(x_ref[...], w_ref[...], preferred_element_type=_f32)


def _scale_body(degacc_ref, xw_ref, dis_ref, y_ref):
    deg = degacc_ref[0] + degacc_ref[1] + 1.0
    dis = lax.rsqrt(deg)
    dis_ref[...] = dis
    y_ref[...] = xw_ref[...] * dis[:, 0:1]


def _layer1_body(acc_ref, y_ref, dis_ref, b_ref, o_ref):
    dis = dis_ref[:, 0:1]
    agg = dis * (acc_ref[0] + acc_ref[1] + y_ref[...])
    h = jnp.maximum(agg + b_ref[...], 0.0)
    o_ref[...] = dis * h


def _layer2_body(acc_ref, y_ref, dis_ref, w_ref, b_ref, o_ref):
    dis = dis_ref[:, 0:1]
    agg = dis * (acc_ref[0] + acc_ref[1] + y_ref[...])
    z = jnp.dot(agg, w_ref[...], preferred_element_type=_f32) + b_ref[...]
    o_ref[...] = jax.nn.sigmoid(z)


def kernel(x, edge_index, W1, b1, W2, b2):
    edge_r = edge_index.reshape(2, ECH, K)
    x_p = jnp.pad(x, ((0, NP - N), (0, 0)))
    zeros16 = jnp.zeros((NP, 16), _f32)
    ones16 = jnp.ones((K, 16), _f32)
    zeros32 = jnp.zeros((NP, DH), _f32)
    b1r = b1.reshape(1, DH)
    b2r = b2.reshape(1, D_OUT)

    degacc = _sc_degree(edge_r, zeros16, ones16)
    xw = pl.pallas_call(
        _mm1_body,
        out_shape=jax.ShapeDtypeStruct((NP, DH), _f32),
    )(x_p, W1)

    dis, y1 = pl.pallas_call(
        _scale_body,
        out_shape=[jax.ShapeDtypeStruct((NP, 16), _f32),
                   jax.ShapeDtypeStruct((NP, DH), _f32)],
    )(degacc, xw)

    acc1 = _sc_aggregate(y1, edge_r, zeros32)

    y2 = pl.pallas_call(
        _layer1_body,
        out_shape=jax.ShapeDtypeStruct((NP, DH), _f32),
    )(acc1, y1, dis, b1r)

    acc2 = _sc_aggregate(y2, edge_r, zeros32)

    out = pl.pallas_call(
        _layer2_body,
        out_shape=jax.ShapeDtypeStruct((NP, D_OUT), _f32),
    )(acc2, y2, dis, W2, b2r)

    return out[:N]
